# baseline (speedup 1.0000x reference)
import jax
import jax.numpy as jnp
from jax import lax
from jax.experimental import pallas as pl
from jax.experimental.pallas import tpu as pltpu
from jax.experimental.pallas import tpu_sc as plsc

N = 10000
NP = 10240
E = 320000
C = 128
H = 64
NLAYERS = 3
ALPHA = 0.9
LASTC = 1.0 - ALPHA

NCORE = 2
NSUB = 16
RT = NP // NSUB
K = 80
ECH = E // NSUB // K
RCH = RT // K
NHIST = 4


def _zero16():
    return jnp.zeros((16,), jnp.float32)


def _body(lab_ref, src_ref, dst_ref, mask_ref, y_ref,
          srcbuf, dstbuf, rowsA, rowsB, rowsC, rowsD, obufA, obufB,
          zbuf, hist, normbuf, mbuf, agg, hist_all, h_hbm,
          gsemA, gsemB, gsemC, gsemD, ssemA, ssemB, ssemC, ssemD):
    rows = (rowsA, rowsB, rowsC, rowsD)
    obuf = (obufA, obufB)
    gsem = (gsemA, gsemB, gsemC, gsemD)
    ssem = (ssemA, ssemB, ssemC, ssemD)

    c = lax.axis_index("c")
    s = lax.axis_index("s")
    row0 = s * RT
    ebase = s * ECH
    coff = c * NP

    @pl.loop(0, K)
    def _(r):
        for q in range(4):
            zbuf[r, pl.ds(q * 16, 16)] = _zero16()

    @pl.loop(0, NP // 16)
    def _(i):
        hist[pl.ds(i * 16, 16)] = _zero16()

    pltpu.sync_copy(src_ref.at[pl.ds(ebase, ECH), :], srcbuf)
    pltpu.sync_copy(dst_ref.at[pl.ds(ebase, ECH), :], dstbuf)

    coffv = jnp.full((16,), coff, jnp.int32)

    @pl.loop(0, ECH)
    def _(r):
        for q in range(5):
            srcbuf[r, pl.ds(q * 16, 16)] = srcbuf[r, pl.ds(q * 16, 16)] + coffv

    ones16 = jnp.ones((16,), jnp.float32)

    @pl.loop(0, ECH)
    def _(r):
        for q in range(5):
            plsc.addupdate_scatter(hist, [dstbuf[r, pl.ds(q * 16, 16)]], ones16)

    plsc.subcore_barrier()

    pltpu.sync_copy(hist, hist_all.at[s])
    plsc.subcore_barrier()

    pltpu.sync_copy(hist_all.at[0, pl.ds(row0, RT)], normbuf)
    for t2 in range(1, NSUB):
        pltpu.sync_copy(hist_all.at[t2, pl.ds(row0, RT)], mbuf)

        @pl.loop(0, RT // 16)
        def _(i):
            sl = pl.ds(i * 16, 16)
            normbuf[sl] = normbuf[sl] + mbuf[sl]

    @pl.loop(0, RT // 16)
    def _(i):
        d = jnp.maximum(normbuf[pl.ds(i * 16, 16)], 1.0)
        xi = lax.bitcast_convert_type(d, jnp.int32)
        xi = 0x5F3759DF - lax.shift_right_arithmetic(xi, 1)
        yv = lax.bitcast_convert_type(xi, jnp.float32)
        for _ in range(3):
            yv = yv * (1.5 - 0.5 * d * yv * yv)
        normbuf[pl.ds(i * 16, 16)] = yv

    pltpu.sync_copy(mask_ref.at[pl.ds(row0, RT)], mbuf)

    @pl.loop(0, RCH)
    def _(u):
        gbase = row0 + u * K
        pltpu.sync_copy(zbuf, agg.at[pl.ds(gbase, K), :])

        @pl.when(gbase + K <= N)
        def _():
            pltpu.sync_copy(
                lab_ref.at[pl.ds(gbase, K), pl.ds(c * H, H)], rowsA)

            @pl.loop(0, K // 16)
            def _(g):
                lbase = u * K + g * 16
                mv = mbuf[pl.ds(lbase, 16)]
                nv = normbuf[pl.ds(lbase, 16)]
                for j in range(16):
                    r = g * 16 + j
                    mn = mv[j] * nv[j]
                    for q in range(4):
                        sl = pl.ds(q * 16, 16)
                        rowsA[r, sl] = rowsA[r, sl] * mn

            pltpu.sync_copy(rowsA, h_hbm.at[pl.ds(coff + gbase, K), :])

        @pl.when(gbase + K > N)
        def _():
            pltpu.sync_copy(zbuf, h_hbm.at[pl.ds(coff + gbase, K), :])

    plsc.subcore_barrier()

    for layer in range(NLAYERS):
        final = layer == NLAYERS - 1

        def wait_g(i, b):
            pltpu.make_async_copy(h_hbm.at[srcbuf.at[i]], rows[b], gsem[b]).wait()

        def start_g(i, b):
            pltpu.async_copy(h_hbm.at[srcbuf.at[i]], rows[b], gsem[b])

        def start_s(i, b):
            pltpu.async_copy(rows[b], agg.at[dstbuf.at[i]], ssem[b], add=True)

        def wait_s(i, b):
            pltpu.make_async_copy(rows[b], agg.at[dstbuf.at[i]], ssem[b]).wait()

        start_g(0, 0)
        start_g(1, 1)
        for i in range(4):
            b = i % 4
            wait_g(i, b)
            start_s(i, b)
            b2 = (i + 2) % 4
            if i >= 2:
                wait_s(i - 2, b2)
            start_g(i + 2, b2)

        @pl.loop(0, (ECH - 6) // 4)
        def _(o):
            for b in range(4):
                i = 4 + o * 4 + b
                wait_g(i, b)
                start_s(i, b)
                b2 = (b + 2) % 4
                wait_s(i - 2, b2)
                start_g(i + 2, b2)

        for i in range(ECH - 2, ECH):
            b = i % 4
            wait_g(i, b)
            start_s(i, b)
        for i in range(ECH - 4, ECH):
            wait_s(i, i % 4)

        plsc.subcore_barrier()

        def valid(u):
            return row0 + u * K + K <= N

        def agg_sl(u):
            return agg.at[pl.ds(row0 + u * K, K), :]

        def lab_sl(u):
            return lab_ref.at[pl.ds(row0 + u * K, K), pl.ds(c * H, H)]

        def out_sl(u):
            if final:
                return y_ref.at[pl.ds(row0 + u * K, K), pl.ds(c * H, H)]
            return h_hbm.at[pl.ds(coff + row0 + u * K, K), :]

        pltpu.async_copy(agg_sl(0), rows[0], gsem[0])
        pltpu.async_copy(agg_sl(1), rows[1], gsem[1])
        for u in range(2):
            @pl.when(valid(u))
            def _(u=u):
                pltpu.async_copy(lab_sl(u), rows[2 + u], gsem[2 + u])

        @pl.loop(0, RCH // 2)
        def _(o):
            for p in range(2):
                u = o * 2 + p
                pltpu.make_async_copy(agg_sl(u), rows[p], gsem[p]).wait()
                if not final:
                    pltpu.sync_copy(zbuf, agg_sl(u))

                @pl.when(valid(u))
                def _(u=u, p=p):
                    pltpu.make_async_copy(
                        lab_sl(u), rows[2 + p], gsem[2 + p]).wait()

                    @pl.loop(0, K // 16)
                    def _(g):
                        lbase = u * K + g * 16
                        mv = mbuf[pl.ds(lbase, 16)]
                        nv = normbuf[pl.ds(lbase, 16)]
                        for j in range(16):
                            r = g * 16 + j
                            lm = LASTC * mv[j]
                            nm = nv[j]
                            for q in range(4):
                                sl = pl.ds(q * 16, 16)
                                yv = (lm * rows[2 + p][r, sl]
                                      + ALPHA * rows[p][r, sl] * nm)
                                yv = jnp.minimum(jnp.maximum(yv, 0.0), 1.0)
                                obuf[p][r, sl] = yv if final else yv * nm

                    pltpu.sync_copy(obuf[p], out_sl(u))

                @pl.when(u + 2 < RCH)
                def _(u=u, p=p):
                    pltpu.async_copy(agg_sl(u + 2), rows[p], gsem[p])

                    @pl.when(valid(u + 2))
                    def _():
                        pltpu.async_copy(lab_sl(u + 2), rows[2 + p], gsem[2 + p])

        plsc.subcore_barrier()


@jax.jit
def _run(lab, src2d, dst2d, mask_f):
    mesh = plsc.VectorSubcoreMesh(
        core_axis_name="c", subcore_axis_name="s",
        num_cores=NCORE, num_subcores=NSUB)
    kern = pl.kernel(
        _body,
        out_type=jax.ShapeDtypeStruct((N, C), jnp.float32),
        mesh=mesh,
        compiler_params=pltpu.CompilerParams(
            use_tc_tiling_on_sc=False, needs_layout_passes=False),
        scratch_types=[
            pltpu.VMEM((ECH, K), jnp.int32),
            pltpu.VMEM((ECH, K), jnp.int32),
            pltpu.VMEM((K, H), jnp.float32),
            pltpu.VMEM((K, H), jnp.float32),
            pltpu.VMEM((K, H), jnp.float32),
            pltpu.VMEM((K, H), jnp.float32),
            pltpu.VMEM((K, H), jnp.float32),
            pltpu.VMEM((K, H), jnp.float32),
            pltpu.VMEM((K, H), jnp.float32),
            pltpu.VMEM((NP,), jnp.float32),
            pltpu.VMEM((RT,), jnp.float32),
            pltpu.VMEM((RT,), jnp.float32),
            pltpu.VMEM_SHARED((NP, H), jnp.float32),
            pltpu.HBM((NSUB, NP), jnp.float32),
            pltpu.HBM((NCORE * NP, H), jnp.float32),
        ] + [pltpu.SemaphoreType.DMA] * 8,
    )
    return kern(lab, src2d, dst2d, mask_f)


def kernel(labels, edge_index, mask):
    mask_f = jnp.zeros((NP,), jnp.float32).at[:N].set(mask.astype(jnp.float32))
    src2d = edge_index[0].reshape(E // K, K)
    dst2d = edge_index[1].reshape(E // K, K)
    return _run(labels, src2d, dst2d, mask_f)
